# baseline (device time: 45398 ns/iter reference)
import jax
import jax.numpy as jnp
from jax import lax
from jax.experimental import pallas as pl
from jax.experimental.pallas import tpu as pltpu

T = 1024
D = 1024
F = 2048
E_LOC = 2
N_EXP = 4
C = 288
CH = 1024
B = 128


def kernel(x, assign, W1, W2):
    assign2d = assign.reshape(T, 1)

    def body(x_ref, a_ref, w1_any, w2_any, out_ref,
             xps, xrecv, osend, orecv, w1b, w2b, stg,
             fsend_sems, frecv_sems, rsend_sems, rrecv_sems,
             ol_send, ol_recv, ssw_sems, rsw_sems, wsems):
        my_x = lax.axis_index("x")
        peer = (1 - my_x, lax.axis_index("y"))

        WPLAN = [(0, 0, 0), (0, 1, 0), (0, 0, 1), (0, 1, 1),
                 (1, 0, 0), (1, 1, 0), (1, 0, 1), (1, 1, 1)]
        NW = len(WPLAN)
        NSTG = 4

        def wchunk_copy(k):
            j, h, m = WPLAN[k]
            src = (w1_any.at[j, :, pl.ds(h * CH, CH)] if m == 0
                   else w2_any.at[j, pl.ds(h * CH, CH), :])
            return pltpu.make_async_copy(src, stg.at[k % NSTG], wsems.at[k])

        def wchunk_cast(k):
            j, h, m = WPLAN[k]
            val = stg[k % NSTG].astype(jnp.bfloat16)
            if m == 0:
                w1b[j, :, pl.ds(h * CH, CH)] = val
            else:
                w2b[j, pl.ds(h * CH, CH), :] = val

        wdmas = {k: wchunk_copy(k) for k in range(NW)}
        for k in range(NSTG):
            wdmas[k].start()

        partner = (my_x, 1 - lax.axis_index("y"))
        barrier = pltpu.get_barrier_semaphore()
        for nbr in (peer, partner):
            pl.semaphore_signal(barrier, inc=1, device_id=nbr,
                                device_id_type=pl.DeviceIdType.MESH)
        pl.semaphore_wait(barrier, 2)

        a = a_ref[...]
        e_iota = lax.broadcasted_iota(jnp.int32, (T, N_EXP), 1)
        e1 = (a == e_iota).astype(jnp.bfloat16)
        tri_b = (lax.broadcasted_iota(jnp.int32, (B, B), 0)
                 > lax.broadcasted_iota(jnp.int32, (B, B), 1)
                 ).astype(jnp.bfloat16)
        rank4_parts = []
        off = jnp.zeros((1, N_EXP), jnp.float32)
        for b in range(T // B):
            blk = e1[b * B:(b + 1) * B]
            intra = jnp.dot(tri_b, blk,
                            preferred_element_type=jnp.float32)
            rank4_parts.append(intra + off)
            off = off + jnp.sum(blk, axis=0, keepdims=True,
                                dtype=jnp.float32)
        rank4 = jnp.concatenate(rank4_parts, axis=0)
        rank = jnp.sum(rank4 * e1.astype(jnp.float32), axis=1,
                       keepdims=True).astype(jnp.int32)

        pos = jnp.remainder(a - E_LOC * my_x, N_EXP)
        valid = rank < C
        h_iota = lax.broadcasted_iota(jnp.int32, (T, 2 * C), 1)
        slot_r = jnp.where(valid & (pos >= 2), (pos - 2) * C + rank, 2 * C)
        p_rem = (slot_r == h_iota).astype(jnp.bfloat16)

        xb = x_ref[...].astype(jnp.bfloat16)

        def pack(p_half, j):
            return lax.dot_general(
                p_half[:, j * C:(j + 1) * C], xb,
                (((0,), (0,)), ((), ())),
                preferred_element_type=jnp.float32).astype(jnp.bfloat16)

        rdma_f = []
        for j in range(E_LOC):
            xps[pl.ds((2 + j) * C, C), :] = pack(p_rem, j)
            r = pltpu.make_async_remote_copy(
                src_ref=xps.at[pl.ds((2 + j) * C, C), :],
                dst_ref=xrecv.at[pl.ds(j * C, C), :],
                send_sem=fsend_sems.at[j], recv_sem=frecv_sems.at[j],
                device_id=peer, device_id_type=pl.DeviceIdType.MESH)
            r.start()
            rdma_f.append(r)

        slot_l = jnp.where(valid & (pos < 2), pos * C + rank, 2 * C)
        p_loc = (slot_l == h_iota).astype(jnp.bfloat16)
        for j in range(E_LOC):
            xps[pl.ds(j * C, C), :] = pack(p_loc, j)

        def wdrain(k):
            wdmas[k].wait()
            wchunk_cast(k)
            if k + NSTG < NW:
                wdmas[k + NSTG].start()

        for k in range(4):
            wdrain(k)

        def ffn(xblk, j):
            h = jnp.maximum(
                jnp.dot(xblk, w1b[j], preferred_element_type=jnp.float32),
                0.0).astype(jnp.bfloat16)
            return jnp.dot(h, w2b[j], preferred_element_type=jnp.float32)

        wdrain(4)
        wdrain(5)

        rdma_r = []
        for j in range(E_LOC):
            rdma_f[j].wait()
            osend[pl.ds(j * C, C), :] = ffn(
                xrecv[pl.ds(j * C, C), :], j).astype(jnp.bfloat16)
            r = pltpu.make_async_remote_copy(
                src_ref=osend.at[pl.ds(j * C, C), :],
                dst_ref=orecv.at[pl.ds(j * C, C), :],
                send_sem=rsend_sems.at[j], recv_sem=rrecv_sems.at[j],
                device_id=peer, device_id_type=pl.DeviceIdType.MESH)
            r.start()
            rdma_r.append(r)
            if j == 0:
                wdrain(6)
                wdrain(7)

        my_y = lax.axis_index("y")

        @pl.when(my_y == 0)
        def _():
            ol_send[...] = ffn(xps[pl.ds(0, C), :], 0).astype(jnp.bfloat16)

        @pl.when(my_y == 1)
        def _():
            ol_send[...] = ffn(xps[pl.ds(C, C), :], 1).astype(jnp.bfloat16)

        swap = pltpu.make_async_remote_copy(
            src_ref=ol_send, dst_ref=ol_recv,
            send_sem=ssw_sems.at[0], recv_sem=rsw_sems.at[0],
            device_id=partner, device_id_type=pl.DeviceIdType.MESH)
        swap.start()

        @pl.when(my_y == 0)
        def _():
            out_ref[...] = jnp.dot(p_loc[:, :C], ol_send[...],
                                   preferred_element_type=jnp.float32)

        @pl.when(my_y == 1)
        def _():
            out_ref[...] = jnp.dot(p_loc[:, C:], ol_send[...],
                                   preferred_element_type=jnp.float32)

        for j in range(E_LOC):
            rdma_r[j].wait()
            out_ref[...] = out_ref[...] + jnp.dot(
                p_rem[:, j * C:(j + 1) * C], orecv[pl.ds(j * C, C), :],
                preferred_element_type=jnp.float32)

        swap.wait()

        @pl.when(my_y == 0)
        def _():
            out_ref[...] = out_ref[...] + jnp.dot(
                p_loc[:, C:], ol_recv[...],
                preferred_element_type=jnp.float32)

        @pl.when(my_y == 1)
        def _():
            out_ref[...] = out_ref[...] + jnp.dot(
                p_loc[:, :C], ol_recv[...],
                preferred_element_type=jnp.float32)

    return pl.pallas_call(
        body,
        out_shape=jax.ShapeDtypeStruct((T, D), jnp.float32),
        in_specs=[
            pl.BlockSpec(memory_space=pltpu.VMEM),
            pl.BlockSpec(memory_space=pltpu.VMEM),
            pl.BlockSpec(memory_space=pltpu.MemorySpace.HBM),
            pl.BlockSpec(memory_space=pltpu.MemorySpace.HBM),
        ],
        out_specs=pl.BlockSpec(memory_space=pltpu.VMEM),
        scratch_shapes=[
            pltpu.VMEM((N_EXP * C, D), jnp.bfloat16),
            pltpu.VMEM((2 * C, D), jnp.bfloat16),
            pltpu.VMEM((2 * C, D), jnp.bfloat16),
            pltpu.VMEM((2 * C, D), jnp.bfloat16),
            pltpu.VMEM((E_LOC, D, F), jnp.bfloat16),
            pltpu.VMEM((E_LOC, F, D), jnp.bfloat16),
            pltpu.VMEM((4, CH, CH), jnp.float32),
            pltpu.SemaphoreType.DMA((2,)),
            pltpu.SemaphoreType.DMA((2,)),
            pltpu.SemaphoreType.DMA((2,)),
            pltpu.SemaphoreType.DMA((2,)),
            pltpu.VMEM((C, D), jnp.bfloat16),
            pltpu.VMEM((C, D), jnp.bfloat16),
            pltpu.SemaphoreType.DMA((1,)),
            pltpu.SemaphoreType.DMA((1,)),
            pltpu.SemaphoreType.DMA((8,)),
        ],
        compiler_params=pltpu.CompilerParams(
            collective_id=0, vmem_limit_bytes=100 * 1024 * 1024),
    )(x, assign2d, W1, W2)


# device time: 41351 ns/iter; 1.0979x vs baseline; 1.0979x over previous
import jax
import jax.numpy as jnp
from jax import lax
from jax.experimental import pallas as pl
from jax.experimental.pallas import tpu as pltpu

T = 1024
D = 1024
F = 2048
E_LOC = 2
N_EXP = 4
C = 272
CH = 1024
B = 128


def kernel(x, assign, W1, W2):
    assign2d = assign.reshape(T, 1)

    def body(x_ref, a_ref, w1_any, w2_any, out_ref,
             xps, xrecv, osend, orecv, w1b, w2b, stg,
             fsend_sems, frecv_sems, rsend_sems, rrecv_sems, wsems):
        my_x = lax.axis_index("x")
        peer = (1 - my_x, lax.axis_index("y"))

        WPLAN = [(0, 0, 0), (0, 1, 0), (0, 0, 1), (0, 1, 1),
                 (1, 0, 0), (1, 1, 0), (1, 0, 1), (1, 1, 1)]
        NW = len(WPLAN)
        NSTG = 4

        def wchunk_copy(k):
            j, h, m = WPLAN[k]
            src = (w1_any.at[j, :, pl.ds(h * CH, CH)] if m == 0
                   else w2_any.at[j, pl.ds(h * CH, CH), :])
            return pltpu.make_async_copy(src, stg.at[k % NSTG], wsems.at[k])

        def wchunk_cast(k):
            j, h, m = WPLAN[k]
            val = stg[k % NSTG].astype(jnp.bfloat16)
            if m == 0:
                w1b[j, :, pl.ds(h * CH, CH)] = val
            else:
                w2b[j, pl.ds(h * CH, CH), :] = val

        wdmas = {k: wchunk_copy(k) for k in range(NW)}
        for k in range(NSTG):
            wdmas[k].start()

        barrier = pltpu.get_barrier_semaphore()
        pl.semaphore_signal(barrier, inc=1, device_id=peer,
                            device_id_type=pl.DeviceIdType.MESH)
        pl.semaphore_wait(barrier, 1)

        a = a_ref[...]
        e_iota = lax.broadcasted_iota(jnp.int32, (T, N_EXP), 1)
        e1 = (a == e_iota).astype(jnp.bfloat16)
        tri_b = (lax.broadcasted_iota(jnp.int32, (B, B), 0)
                 > lax.broadcasted_iota(jnp.int32, (B, B), 1)
                 ).astype(jnp.bfloat16)
        rank4_parts = []
        off = jnp.zeros((1, N_EXP), jnp.float32)
        for b in range(T // B):
            blk = e1[b * B:(b + 1) * B]
            intra = jnp.dot(tri_b, blk,
                            preferred_element_type=jnp.float32)
            rank4_parts.append(intra + off)
            off = off + jnp.sum(blk, axis=0, keepdims=True,
                                dtype=jnp.float32)
        rank4 = jnp.concatenate(rank4_parts, axis=0)
        rank = jnp.sum(rank4 * e1.astype(jnp.float32), axis=1,
                       keepdims=True).astype(jnp.int32)

        pos = jnp.remainder(a - E_LOC * my_x, N_EXP)
        valid = rank < C
        h_iota = lax.broadcasted_iota(jnp.int32, (T, 2 * C), 1)
        slot_r = jnp.where(valid & (pos >= 2), (pos - 2) * C + rank, 2 * C)
        p_rem = (slot_r == h_iota).astype(jnp.bfloat16)

        xb = x_ref[...].astype(jnp.bfloat16)

        def pack(p_half, j):
            return lax.dot_general(
                p_half[:, j * C:(j + 1) * C], xb,
                (((0,), (0,)), ((), ())),
                preferred_element_type=jnp.float32).astype(jnp.bfloat16)

        rdma_f = []
        for j in range(E_LOC):
            xps[pl.ds((2 + j) * C, C), :] = pack(p_rem, j)
            r = pltpu.make_async_remote_copy(
                src_ref=xps.at[pl.ds((2 + j) * C, C), :],
                dst_ref=xrecv.at[pl.ds(j * C, C), :],
                send_sem=fsend_sems.at[j], recv_sem=frecv_sems.at[j],
                device_id=peer, device_id_type=pl.DeviceIdType.MESH)
            r.start()
            rdma_f.append(r)

        slot_l = jnp.where(valid & (pos < 2), pos * C + rank, 2 * C)
        p_loc = (slot_l == h_iota).astype(jnp.bfloat16)
        for j in range(E_LOC):
            xps[pl.ds(j * C, C), :] = pack(p_loc, j)

        def wdrain(k):
            wdmas[k].wait()
            wchunk_cast(k)
            if k + NSTG < NW:
                wdmas[k + NSTG].start()

        for k in range(4):
            wdrain(k)

        def ffn(xblk, j):
            h = jnp.maximum(
                jnp.dot(xblk, w1b[j], preferred_element_type=jnp.float32),
                0.0).astype(jnp.bfloat16)
            return jnp.dot(h, w2b[j], preferred_element_type=jnp.float32)

        oloc0 = ffn(xps[pl.ds(0, C), :], 0).astype(jnp.bfloat16)

        wdrain(4)
        wdrain(5)

        rdma_r = []
        for j in range(E_LOC):
            rdma_f[j].wait()
            osend[pl.ds(j * C, C), :] = ffn(
                xrecv[pl.ds(j * C, C), :], j).astype(jnp.bfloat16)
            r = pltpu.make_async_remote_copy(
                src_ref=osend.at[pl.ds(j * C, C), :],
                dst_ref=orecv.at[pl.ds(j * C, C), :],
                send_sem=rsend_sems.at[j], recv_sem=rrecv_sems.at[j],
                device_id=peer, device_id_type=pl.DeviceIdType.MESH)
            r.start()
            rdma_r.append(r)
            if j == 0:
                wdrain(6)
                wdrain(7)

        oloc1 = ffn(xps[pl.ds(C, C), :], 1).astype(jnp.bfloat16)
        oloc = jnp.concatenate([oloc0, oloc1], axis=0)
        out_ref[...] = jnp.dot(p_loc, oloc,
                               preferred_element_type=jnp.float32)

        for j in range(E_LOC):
            rdma_r[j].wait()
            out_ref[...] = out_ref[...] + jnp.dot(
                p_rem[:, j * C:(j + 1) * C], orecv[pl.ds(j * C, C), :],
                preferred_element_type=jnp.float32)

    return pl.pallas_call(
        body,
        out_shape=jax.ShapeDtypeStruct((T, D), jnp.float32),
        in_specs=[
            pl.BlockSpec(memory_space=pltpu.VMEM),
            pl.BlockSpec(memory_space=pltpu.VMEM),
            pl.BlockSpec(memory_space=pltpu.MemorySpace.HBM),
            pl.BlockSpec(memory_space=pltpu.MemorySpace.HBM),
        ],
        out_specs=pl.BlockSpec(memory_space=pltpu.VMEM),
        scratch_shapes=[
            pltpu.VMEM((N_EXP * C, D), jnp.bfloat16),
            pltpu.VMEM((2 * C, D), jnp.bfloat16),
            pltpu.VMEM((2 * C, D), jnp.bfloat16),
            pltpu.VMEM((2 * C, D), jnp.bfloat16),
            pltpu.VMEM((E_LOC, D, F), jnp.bfloat16),
            pltpu.VMEM((E_LOC, F, D), jnp.bfloat16),
            pltpu.VMEM((4, CH, CH), jnp.float32),
            pltpu.SemaphoreType.DMA((2,)),
            pltpu.SemaphoreType.DMA((2,)),
            pltpu.SemaphoreType.DMA((2,)),
            pltpu.SemaphoreType.DMA((2,)),
            pltpu.SemaphoreType.DMA((8,)),
        ],
        compiler_params=pltpu.CompilerParams(
            collective_id=0, vmem_limit_bytes=100 * 1024 * 1024),
    )(x, assign2d, W1, W2)
